# baseline (speedup 1.0000x reference)
import functools

import jax
import jax.numpy as jnp
import numpy as np
from jax.experimental import pallas as pl
from jax.experimental.pallas import tpu as pltpu

H, L, E = 12, 2048, 64
C = 100
CP = 128
BITS = 32
ITERS = 10
TOPK = 32
BQ = 256
NBQ = L // BQ
TEMP = 1.0 / np.sqrt(E).astype(np.float32)


def _cluster_kernel(q_ref, w_ref, b_ref, qg_ref, dest_ref, sq_ref, sindx_ref,
                    sclus_ref):
    q = q_ref[0]
    h = jnp.dot(q.astype(jnp.bfloat16), w_ref[...].astype(jnp.bfloat16),
                preferred_element_type=jnp.float32)
    h = h + b_ref[0:1, :]
    bit = (h > 0).astype(jnp.float32)
    rs_bit = jnp.sum(bit, axis=1, keepdims=True)

    iota_c = jax.lax.broadcasted_iota(jnp.int32, (1, CP), 1)
    iota_cf = iota_c.astype(jnp.float32)
    iota_l = jax.lax.broadcasted_iota(jnp.int32, (CP, L), 1)
    ci = jax.lax.broadcasted_iota(jnp.int32, (CP, 1), 0).astype(jnp.float32)
    init_idx = jnp.floor(ci * float(L) / float(C)).astype(jnp.int32)
    oh_init = (iota_l == init_idx).astype(jnp.bfloat16)
    bitb = bit.astype(jnp.bfloat16)
    bit16 = (16.0 * bit).astype(jnp.bfloat16)
    cbit0 = jnp.dot(oh_init, bitb, preferred_element_type=jnp.float32)
    invalid = (iota_c >= C).astype(jnp.float32) * 1e9
    rs128 = 128.0 * rs_bit

    def dist_labels(cbit):
        rs_c = jnp.sum(cbit, axis=1)[None, :]
        base = 128.0 * rs_c + iota_cf + invalid
        cbit16 = (16.0 * cbit).astype(jnp.bfloat16)
        dot256 = jax.lax.dot_general(bit16, cbit16, (((1,), (1,)), ((), ())),
                                     preferred_element_type=jnp.float32)
        key = (rs128 + base) - dot256
        m = jnp.min(key, axis=1, keepdims=True)
        labf = m - 128.0 * jnp.floor(m * (1.0 / 128.0))
        oh = (iota_cf == labf).astype(jnp.float32)
        return labf, oh

    def body(_, cbit):
        _, oh = dist_labels(cbit)
        cnt = jnp.sum(oh, axis=0)[:, None]
        bcnt = jax.lax.dot_general(oh.astype(jnp.bfloat16), bitb,
                                   (((0,), (0,)), ((), ())),
                                   preferred_element_type=jnp.float32)
        newc = (2.0 * bcnt > cnt).astype(jnp.float32)
        return jnp.where(cnt > 0, newc, cbit)

    cbit = jax.lax.fori_loop(0, ITERS, body, cbit0)
    labf, oh = dist_labels(cbit)
    lab = labf.astype(jnp.int32)
    cnt = jnp.sum(oh, axis=0)[:, None]
    f = 1.0 / jnp.maximum(cnt, 1.0)
    qg = jax.lax.dot_general(oh, q, (((0,), (0,)), ((), ())),
                             preferred_element_type=jnp.float32,
                             precision=jax.lax.Precision.HIGHEST) * f
    qg_ref[0] = qg

    cnt_row = jnp.sum(oh, axis=0, keepdims=True)
    iu_r = jax.lax.broadcasted_iota(jnp.int32, (CP, CP), 0)
    iu_c = jax.lax.broadcasted_iota(jnp.int32, (CP, CP), 1)
    U = (iu_r < iu_c).astype(jnp.bfloat16)
    cnt_hi = jnp.floor(cnt_row / 256.0)
    cnt_lo = cnt_row - 256.0 * cnt_hi
    off_row = (256.0 * jnp.dot(cnt_hi.astype(jnp.bfloat16), U,
                               preferred_element_type=jnp.float32)
               + jnp.dot(cnt_lo.astype(jnp.bfloat16), U,
                         preferred_element_type=jnp.float32))

    CH = 128
    it_r = jax.lax.broadcasted_iota(jnp.int32, (CH, CH), 0)
    it_c = jax.lax.broadcasted_iota(jnp.int32, (CH, CH), 1)
    T = (it_r > it_c).astype(jnp.bfloat16)
    ohb = oh.astype(jnp.bfloat16)
    carry = off_row
    dest_chunks = []
    for t in range(L // CH):
        ohc = oh[t * CH:(t + 1) * CH, :]
        win = jnp.dot(T, ohb[t * CH:(t + 1) * CH, :],
                      preferred_element_type=jnp.float32)
        sel = jnp.sum(ohc * (carry + win), axis=1, keepdims=True)
        dest_chunks.append(sel)
        carry = carry + jnp.sum(ohc, axis=0, keepdims=True)
    dest = jnp.concatenate(dest_chunks, axis=0)
    dest_ref[0] = dest.astype(jnp.int32)

    iota_j = jax.lax.broadcasted_iota(jnp.int32, (L, L), 1)
    OHb = (iota_j == dest.astype(jnp.int32)).astype(jnp.bfloat16)
    sq = jax.lax.dot_general(OHb, q.astype(jnp.bfloat16),
                             (((0,), (0,)), ((), ())),
                             preferred_element_type=jnp.float32)
    sq_ref[0] = sq.astype(jnp.bfloat16)

    ivec = jax.lax.broadcasted_iota(jnp.int32, (L, 1), 0).astype(jnp.float32)
    ihi = jnp.floor(ivec / 256.0)
    ilo = ivec - 256.0 * ihi
    sidx = (256.0 * jax.lax.dot_general(OHb, ihi.astype(jnp.bfloat16),
                                        (((0,), (0,)), ((), ())),
                                        preferred_element_type=jnp.float32)
            + jax.lax.dot_general(OHb, ilo.astype(jnp.bfloat16),
                                  (((0,), (0,)), ((), ())),
                                  preferred_element_type=jnp.float32))
    sindx_ref[0] = sidx.astype(jnp.int32)

    scl = jax.lax.dot_general(OHb, lab.astype(jnp.bfloat16),
                              (((0,), (0,)), ((), ())),
                              preferred_element_type=jnp.float32)
    sclus_ref[0] = scl.astype(jnp.int32)


def _run_cluster(qt, w, b):
    return pl.pallas_call(
        _cluster_kernel,
        grid=(H,),
        in_specs=[
            pl.BlockSpec((1, L, E), lambda h: (h, 0, 0)),
            pl.BlockSpec((E, BITS), lambda h: (0, 0)),
            pl.BlockSpec((8, BITS), lambda h: (0, 0)),
        ],
        out_specs=[
            pl.BlockSpec((1, CP, E), lambda h: (h, 0, 0)),
            pl.BlockSpec((1, L, 1), lambda h: (h, 0, 0)),
            pl.BlockSpec((1, L, E), lambda h: (h, 0, 0)),
            pl.BlockSpec((1, L, 1), lambda h: (h, 0, 0)),
            pl.BlockSpec((1, L, 1), lambda h: (h, 0, 0)),
        ],
        out_shape=[
            jax.ShapeDtypeStruct((H, CP, E), jnp.float32),
            jax.ShapeDtypeStruct((H, L, 1), jnp.int32),
            jax.ShapeDtypeStruct((H, L, E), jnp.bfloat16),
            jax.ShapeDtypeStruct((H, L, 1), jnp.int32),
            jax.ShapeDtypeStruct((H, L, 1), jnp.int32),
        ],
    )(qt, w, b)


def _topk_kernel(qg_ref, kv_ref, mask_ref, topi_ref, gkv_ref):
    qg = qg_ref[0]
    kvb = kv_ref[0].astype(jnp.bfloat16)
    kb = kvb[:, :E]
    s = jax.lax.dot_general(qg.astype(jnp.bfloat16), kb,
                            (((1,), (1,)), ((), ())),
                            preferred_element_type=jnp.float32)
    s = s + mask_ref[0:1, :]
    iota_l = jax.lax.broadcasted_iota(jnp.int32, (CP, L), 1)
    for j in range(TOPK):
        m = jnp.max(s, axis=1, keepdims=True)
        cand = jnp.where(s == m, iota_l, L)
        idx = jnp.min(cand, axis=1, keepdims=True)
        topi_ref[0, :, j:j + 1] = idx
        ohk = (iota_l == idx).astype(jnp.bfloat16)
        gkv_ref[0, j] = jnp.dot(ohk, kvb,
                                preferred_element_type=jnp.float32
                                ).astype(jnp.bfloat16)
        s = jnp.where(iota_l == idx, -jnp.inf, s)


def _run_topk(qg, kv, maskvec):
    return pl.pallas_call(
        _topk_kernel,
        grid=(H,),
        in_specs=[
            pl.BlockSpec((1, CP, E), lambda h: (h, 0, 0)),
            pl.BlockSpec((1, L, 2 * E), lambda h: (h, 0, 0)),
            pl.BlockSpec((8, L), lambda h: (0, 0)),
        ],
        out_specs=[
            pl.BlockSpec((1, CP, TOPK), lambda h: (h, 0, 0)),
            pl.BlockSpec((1, TOPK, CP, 2 * E), lambda h: (h, 0, 0, 0)),
        ],
        out_shape=[
            jax.ShapeDtypeStruct((H, CP, TOPK), jnp.int32),
            jax.ShapeDtypeStruct((H, TOPK, CP, 2 * E), jnp.bfloat16),
        ],
    )(qg, kv, maskvec)


GC = 8
GW = GC * TOPK
NG = CP // GC


def _attn_kernel(cb_ref, sq_ref, spos_ref, sc_ref, topi_ref, gkv_ref,
                 out_ref):
    h = pl.program_id(0)
    b = pl.program_id(1)
    g_lo = cb_ref[h, b, 0] // GC
    g_hi = cb_ref[h, b, 1] // GC
    sqb = sq_ref[0]
    spos = spos_ref[0]
    sc = sc_ref[0]
    colc = jax.lax.broadcasted_iota(jnp.int32, (1, GW), 1) // TOPK
    neg_inf = jnp.float32(-jnp.inf)

    def body(g, acc):
        kkvv = gkv_ref[0, pl.ds(g * GW, GW), :]
        kk = kkvv[:, :E]
        vv = kkvv[:, E:]
        ti = topi_ref[0, pl.ds(g, 1), :]
        s = jax.lax.dot_general(sqb, kk, (((1,), (1,)), ((), ())),
                                preferred_element_type=jnp.float32)
        future = ti > spos
        own = sc == (colc + g * GC)
        s = jnp.where(future, -1e7, s) * TEMP
        s = jnp.where(own, s, neg_inf)
        m = jnp.max(s, axis=1, keepdims=True)
        p = jnp.exp(s - jnp.maximum(m, -1e30))
        den = jnp.sum(p, axis=1, keepdims=True)
        a = p / jnp.maximum(den, 1e-30)
        a = jnp.where(future, 0.0, a)
        return acc + jnp.dot(a.astype(jnp.bfloat16), vv,
                             preferred_element_type=jnp.float32)

    acc = jax.lax.fori_loop(g_lo, g_hi + 1, body,
                            jnp.zeros((BQ, E), jnp.float32))
    out_ref[0] = acc


def _run_attn(cbounds, sq, spos3, sc3, topi_g, gkv):
    grid_spec = pltpu.PrefetchScalarGridSpec(
        num_scalar_prefetch=1,
        grid=(H, NBQ),
        in_specs=[
            pl.BlockSpec((1, BQ, E), lambda h, b, cb: (h, b, 0)),
            pl.BlockSpec((1, BQ, 1), lambda h, b, cb: (h * NBQ + b, 0, 0)),
            pl.BlockSpec((1, BQ, 1), lambda h, b, cb: (h * NBQ + b, 0, 0)),
            pl.BlockSpec((1, NG, GW), lambda h, b, cb: (h, 0, 0)),
            pl.BlockSpec((1, CP * TOPK, 2 * E), lambda h, b, cb: (h, 0, 0)),
        ],
        out_specs=pl.BlockSpec((1, BQ, E), lambda h, b, cb: (h, b, 0)),
    )
    return pl.pallas_call(
        _attn_kernel,
        grid_spec=grid_spec,
        out_shape=jax.ShapeDtypeStruct((H, L, E), jnp.float32),
    )(cbounds, sq, spos3, sc3, topi_g, gkv)


def kernel(queries, keys, values, attn_mask, query_lengths, key_lengths,
           planes):
    qt = jnp.transpose(queries, (0, 2, 1, 3)).reshape(H, L, E)
    kt = jnp.transpose(keys, (0, 2, 1, 3)).reshape(H, L, E)
    vt = jnp.transpose(values, (0, 2, 1, 3)).reshape(H, L, E)
    w = planes[:, :E].T
    b = jnp.broadcast_to(planes[:, E][None, :], (8, BITS)) + 0.0

    qg, dest, sq, sindx, sclus = _run_cluster(qt, w, b)

    maskvec = jnp.where(jnp.arange(L) < key_lengths, 0.0, -1e9)
    maskvec = jnp.broadcast_to(maskvec[None, :].astype(jnp.float32), (8, L)) + 0.0
    kv = jnp.concatenate([kt, vt], axis=2)
    topi, gkv4 = _run_topk(qg, kv, maskvec)
    gkv = gkv4.transpose(0, 2, 1, 3).reshape(H, CP * TOPK, 2 * E)

    sclus2 = sclus[..., 0]
    cb_lo = sclus2[:, ::BQ]
    cb_hi = sclus2[:, BQ - 1::BQ]
    cbounds = jnp.stack([cb_lo, cb_hi], axis=-1).astype(jnp.int32)

    spos3 = sindx.reshape(H * NBQ, BQ, 1)
    sc3 = sclus.reshape(H * NBQ, BQ, 1)
    topi_g = topi.reshape(H, NG, GW)

    out_s = _run_attn(cbounds, sq, spos3, sc3, topi_g, gkv)

    out = jnp.take_along_axis(out_s, dest, axis=1)
    out = jnp.transpose(out.reshape(1, H, L, E), (0, 2, 1, 3))
    causal_ok = attn_mask != 0
    return jnp.where(causal_ok, out, jnp.full_like(out, jnp.nan))


# speedup vs baseline: 1.0515x; 1.0515x over previous
import functools

import jax
import jax.numpy as jnp
import numpy as np
from jax.experimental import pallas as pl
from jax.experimental.pallas import tpu as pltpu

H, L, E = 12, 2048, 64
C = 100
CP = 128
BITS = 32
ITERS = 10
TOPK = 32
BQ = 256
NBQ = L // BQ
TEMP = 1.0 / np.sqrt(E).astype(np.float32)


def _cluster_kernel(q_ref, w_ref, b_ref, qg_ref, dest_ref, sq_ref, sindx_ref,
                    sclus_ref):
    q = q_ref[0]
    h = jnp.dot(q.astype(jnp.bfloat16), w_ref[...].astype(jnp.bfloat16),
                preferred_element_type=jnp.float32)
    h = h + b_ref[0:1, :]
    bit = (h > 0).astype(jnp.float32)
    rs_bit = jnp.sum(bit, axis=1, keepdims=True)

    iota_c = jax.lax.broadcasted_iota(jnp.int32, (1, CP), 1)
    iota_cf = iota_c.astype(jnp.float32)
    iota_l = jax.lax.broadcasted_iota(jnp.int32, (CP, L), 1)
    ci = jax.lax.broadcasted_iota(jnp.int32, (CP, 1), 0).astype(jnp.float32)
    init_idx = jnp.floor(ci * float(L) / float(C)).astype(jnp.int32)
    oh_init = (iota_l == init_idx).astype(jnp.bfloat16)
    bitb = bit.astype(jnp.bfloat16)
    bit16 = (16.0 * bit).astype(jnp.bfloat16)
    cbit0 = jnp.dot(oh_init, bitb, preferred_element_type=jnp.float32)
    invalid = (iota_c >= C).astype(jnp.float32) * 1e9
    rs128 = 128.0 * rs_bit

    def dist_labels(cbit):
        rs_c = jnp.sum(cbit, axis=1)[None, :]
        base = 128.0 * rs_c + iota_cf + invalid
        cbit16 = (16.0 * cbit).astype(jnp.bfloat16)
        dot256 = jax.lax.dot_general(bit16, cbit16, (((1,), (1,)), ((), ())),
                                     preferred_element_type=jnp.float32)
        key = (rs128 + base) - dot256
        m = jnp.min(key, axis=1, keepdims=True)
        labf = m - 128.0 * jnp.floor(m * (1.0 / 128.0))
        oh = (iota_cf == labf).astype(jnp.float32)
        return labf, oh

    def body(_, cbit):
        _, oh = dist_labels(cbit)
        cnt = jnp.sum(oh, axis=0)[:, None]
        bcnt = jax.lax.dot_general(oh.astype(jnp.bfloat16), bitb,
                                   (((0,), (0,)), ((), ())),
                                   preferred_element_type=jnp.float32)
        newc = (2.0 * bcnt > cnt).astype(jnp.float32)
        return jnp.where(cnt > 0, newc, cbit)

    cbit = jax.lax.fori_loop(0, ITERS, body, cbit0)
    labf, oh = dist_labels(cbit)
    lab = labf.astype(jnp.int32)
    cnt = jnp.sum(oh, axis=0)[:, None]
    f = 1.0 / jnp.maximum(cnt, 1.0)
    qg = jax.lax.dot_general(oh, q, (((0,), (0,)), ((), ())),
                             preferred_element_type=jnp.float32,
                             precision=jax.lax.Precision.HIGHEST) * f
    qg_ref[0] = qg

    cnt_row = jnp.sum(oh, axis=0, keepdims=True)
    iu_r = jax.lax.broadcasted_iota(jnp.int32, (CP, CP), 0)
    iu_c = jax.lax.broadcasted_iota(jnp.int32, (CP, CP), 1)
    U = (iu_r < iu_c).astype(jnp.bfloat16)
    cnt_hi = jnp.floor(cnt_row / 256.0)
    cnt_lo = cnt_row - 256.0 * cnt_hi
    off_row = (256.0 * jnp.dot(cnt_hi.astype(jnp.bfloat16), U,
                               preferred_element_type=jnp.float32)
               + jnp.dot(cnt_lo.astype(jnp.bfloat16), U,
                         preferred_element_type=jnp.float32))

    CH = 128
    it_r = jax.lax.broadcasted_iota(jnp.int32, (CH, CH), 0)
    it_c = jax.lax.broadcasted_iota(jnp.int32, (CH, CH), 1)
    T = (it_r > it_c).astype(jnp.bfloat16)
    ohb = oh.astype(jnp.bfloat16)
    carry = off_row
    dest_chunks = []
    for t in range(L // CH):
        ohc = oh[t * CH:(t + 1) * CH, :]
        win = jnp.dot(T, ohb[t * CH:(t + 1) * CH, :],
                      preferred_element_type=jnp.float32)
        sel = jnp.sum(ohc * (carry + win), axis=1, keepdims=True)
        dest_chunks.append(sel)
        carry = carry + jnp.sum(ohc, axis=0, keepdims=True)
    dest = jnp.concatenate(dest_chunks, axis=0)
    dest_ref[0] = dest.astype(jnp.int32)

    iota_j = jax.lax.broadcasted_iota(jnp.int32, (L, L), 1)
    OHb = (iota_j == dest.astype(jnp.int32)).astype(jnp.bfloat16)
    sq = jax.lax.dot_general(OHb, q.astype(jnp.bfloat16),
                             (((0,), (0,)), ((), ())),
                             preferred_element_type=jnp.float32)
    sq_ref[0] = sq.astype(jnp.bfloat16)

    ivec = jax.lax.broadcasted_iota(jnp.int32, (L, 1), 0).astype(jnp.float32)
    ihi = jnp.floor(ivec / 256.0)
    ilo = ivec - 256.0 * ihi
    sidx = (256.0 * jax.lax.dot_general(OHb, ihi.astype(jnp.bfloat16),
                                        (((0,), (0,)), ((), ())),
                                        preferred_element_type=jnp.float32)
            + jax.lax.dot_general(OHb, ilo.astype(jnp.bfloat16),
                                  (((0,), (0,)), ((), ())),
                                  preferred_element_type=jnp.float32))
    sindx_ref[0] = sidx.astype(jnp.int32)

    scl = jax.lax.dot_general(OHb, lab.astype(jnp.bfloat16),
                              (((0,), (0,)), ((), ())),
                              preferred_element_type=jnp.float32)
    sclus_ref[0] = scl.astype(jnp.int32)


def _run_cluster(qt, w, b):
    return pl.pallas_call(
        _cluster_kernel,
        grid=(H,),
        in_specs=[
            pl.BlockSpec((1, L, E), lambda h: (h, 0, 0)),
            pl.BlockSpec((E, BITS), lambda h: (0, 0)),
            pl.BlockSpec((8, BITS), lambda h: (0, 0)),
        ],
        out_specs=[
            pl.BlockSpec((1, CP, E), lambda h: (h, 0, 0)),
            pl.BlockSpec((1, L, 1), lambda h: (h, 0, 0)),
            pl.BlockSpec((1, L, E), lambda h: (h, 0, 0)),
            pl.BlockSpec((1, L, 1), lambda h: (h, 0, 0)),
            pl.BlockSpec((1, L, 1), lambda h: (h, 0, 0)),
        ],
        out_shape=[
            jax.ShapeDtypeStruct((H, CP, E), jnp.float32),
            jax.ShapeDtypeStruct((H, L, 1), jnp.int32),
            jax.ShapeDtypeStruct((H, L, E), jnp.bfloat16),
            jax.ShapeDtypeStruct((H, L, 1), jnp.int32),
            jax.ShapeDtypeStruct((H, L, 1), jnp.int32),
        ],
    )(qt, w, b)


def _topk_kernel(qg_ref, k_ref, v_ref, mask_ref, topi_ref, gk_ref, gv_ref):
    qg = qg_ref[0]
    k = k_ref[0]
    kb = k.astype(jnp.bfloat16)
    vb = v_ref[0].astype(jnp.bfloat16)
    s = jax.lax.dot_general(qg.astype(jnp.bfloat16), kb,
                            (((1,), (1,)), ((), ())),
                            preferred_element_type=jnp.float32)
    s = s + mask_ref[0:1, :]
    iota_l = jax.lax.broadcasted_iota(jnp.int32, (CP, L), 1)
    for j in range(TOPK):
        m = jnp.max(s, axis=1, keepdims=True)
        cand = jnp.where(s == m, iota_l, L)
        idx = jnp.min(cand, axis=1, keepdims=True)
        topi_ref[0, :, j:j + 1] = idx
        ohk = (iota_l == idx).astype(jnp.bfloat16)
        gk_ref[0, j] = jnp.dot(ohk, kb,
                               preferred_element_type=jnp.float32
                               ).astype(jnp.bfloat16)
        gv_ref[0, j] = jnp.dot(ohk, vb,
                               preferred_element_type=jnp.float32
                               ).astype(jnp.bfloat16)
        s = jnp.where(iota_l == idx, -jnp.inf, s)


def _run_topk(qg, kt, vt, maskvec):
    return pl.pallas_call(
        _topk_kernel,
        grid=(H,),
        in_specs=[
            pl.BlockSpec((1, CP, E), lambda h: (h, 0, 0)),
            pl.BlockSpec((1, L, E), lambda h: (h, 0, 0)),
            pl.BlockSpec((1, L, E), lambda h: (h, 0, 0)),
            pl.BlockSpec((8, L), lambda h: (0, 0)),
        ],
        out_specs=[
            pl.BlockSpec((1, CP, TOPK), lambda h: (h, 0, 0)),
            pl.BlockSpec((1, TOPK, CP, E), lambda h: (h, 0, 0, 0)),
            pl.BlockSpec((1, TOPK, CP, E), lambda h: (h, 0, 0, 0)),
        ],
        out_shape=[
            jax.ShapeDtypeStruct((H, CP, TOPK), jnp.int32),
            jax.ShapeDtypeStruct((H, TOPK, CP, E), jnp.bfloat16),
            jax.ShapeDtypeStruct((H, TOPK, CP, E), jnp.bfloat16),
        ],
    )(qg, kt, vt, maskvec)


GC = 8
GW = GC * TOPK
NG = CP // GC


def _attn_kernel(cb_ref, sq_ref, spos_ref, sc_ref, topi_ref, gk_ref, gv_ref,
                 out_ref):
    h = pl.program_id(0)
    b = pl.program_id(1)
    g_lo = cb_ref[h, b, 0] // GC
    g_hi = cb_ref[h, b, 1] // GC
    sqb = sq_ref[0]
    spos = spos_ref[0]
    sc = sc_ref[0]
    colc = jax.lax.broadcasted_iota(jnp.int32, (1, GW), 1) // TOPK
    neg_inf = jnp.float32(-jnp.inf)

    def body(g, acc):
        kk = gk_ref[0, pl.ds(g * GW, GW), :]
        vv = gv_ref[0, pl.ds(g * GW, GW), :]
        ti = topi_ref[0, pl.ds(g, 1), :]
        s = jax.lax.dot_general(sqb, kk, (((1,), (1,)), ((), ())),
                                preferred_element_type=jnp.float32)
        future = ti > spos
        own = sc == (colc + g * GC)
        s = jnp.where(future, -1e7, s) * TEMP
        s = jnp.where(own, s, neg_inf)
        m = jnp.max(s, axis=1, keepdims=True)
        p = jnp.exp(s - jnp.maximum(m, -1e30))
        den = jnp.sum(p, axis=1, keepdims=True)
        a = p / jnp.maximum(den, 1e-30)
        a = jnp.where(future, 0.0, a)
        return acc + jnp.dot(a.astype(jnp.bfloat16), vv,
                             preferred_element_type=jnp.float32)

    acc = jax.lax.fori_loop(g_lo, g_hi + 1, body,
                            jnp.zeros((BQ, E), jnp.float32))
    out_ref[0] = acc


def _run_attn(cbounds, sq, spos3, sc3, topi_g, gk, gv):
    grid_spec = pltpu.PrefetchScalarGridSpec(
        num_scalar_prefetch=1,
        grid=(H, NBQ),
        in_specs=[
            pl.BlockSpec((1, BQ, E), lambda h, b, cb: (h, b, 0)),
            pl.BlockSpec((1, BQ, 1), lambda h, b, cb: (h * NBQ + b, 0, 0)),
            pl.BlockSpec((1, BQ, 1), lambda h, b, cb: (h * NBQ + b, 0, 0)),
            pl.BlockSpec((1, NG, GW), lambda h, b, cb: (h, 0, 0)),
            pl.BlockSpec((1, CP * TOPK, E), lambda h, b, cb: (h, 0, 0)),
            pl.BlockSpec((1, CP * TOPK, E), lambda h, b, cb: (h, 0, 0)),
        ],
        out_specs=pl.BlockSpec((1, BQ, E), lambda h, b, cb: (h, b, 0)),
    )
    return pl.pallas_call(
        _attn_kernel,
        grid_spec=grid_spec,
        out_shape=jax.ShapeDtypeStruct((H, L, E), jnp.float32),
    )(cbounds, sq, spos3, sc3, topi_g, gk, gv)


def kernel(queries, keys, values, attn_mask, query_lengths, key_lengths,
           planes):
    qt = jnp.transpose(queries, (0, 2, 1, 3)).reshape(H, L, E)
    kt = jnp.transpose(keys, (0, 2, 1, 3)).reshape(H, L, E)
    vt = jnp.transpose(values, (0, 2, 1, 3)).reshape(H, L, E)
    w = planes[:, :E].T
    b = jnp.broadcast_to(planes[:, E][None, :], (8, BITS)) + 0.0

    qg, dest, sq, sindx, sclus = _run_cluster(qt, w, b)

    maskvec = jnp.where(jnp.arange(L) < key_lengths, 0.0, -1e9)
    maskvec = jnp.broadcast_to(maskvec[None, :].astype(jnp.float32), (8, L)) + 0.0
    topi, gk4, gv4 = _run_topk(qg, kt, vt, maskvec)
    gk = gk4.transpose(0, 2, 1, 3).reshape(H, CP * TOPK, E)
    gv = gv4.transpose(0, 2, 1, 3).reshape(H, CP * TOPK, E)

    sclus2 = sclus[..., 0]
    cb_lo = sclus2[:, ::BQ]
    cb_hi = sclus2[:, BQ - 1::BQ]
    cbounds = jnp.stack([cb_lo, cb_hi], axis=-1).astype(jnp.int32)

    spos3 = sindx.reshape(H * NBQ, BQ, 1)
    sc3 = sclus.reshape(H * NBQ, BQ, 1)
    topi_g = topi.reshape(H, NG, GW)

    out_s = _run_attn(cbounds, sq, spos3, sc3, topi_g, gk, gv)

    out = jnp.take_along_axis(out_s, dest, axis=1)
    out = jnp.transpose(out.reshape(1, H, L, E), (0, 2, 1, 3))
    causal_ok = attn_mask != 0
    return jnp.where(causal_ok, out, jnp.full_like(out, jnp.nan))
